# baseline (speedup 1.0000x reference)
import functools

import jax
import jax.numpy as jnp
from jax import lax
from jax.experimental import pallas as pl
from jax.experimental.pallas import tpu as pltpu
from jax.experimental.pallas import tpu_sc as plsc

_NW = 32
_NB_FULL = 7812
_EDGE_ROWS = _NB_FULL * 128


def _iota16():
    return lax.iota(jnp.int32, 16)


_mesh = plsc.VectorSubcoreMesh(core_axis_name="c", subcore_axis_name="s")


@functools.partial(
    pl.kernel,
    mesh=_mesh,
    out_type=jax.ShapeDtypeStruct((500000, 128), jnp.float32),
    compiler_params=pltpu.CompilerParams(needs_layout_passes=False),
    scratch_types=[
        pltpu.VMEM((64, 128), jnp.float32),
        pltpu.VMEM((64, 128), jnp.float32),
    ],
)
def _transpose(table_t, edge, t2, blk_v, tr_v):
    wid = lax.axis_index("s") * 2 + lax.axis_index("c")

    def block_body(k, carry):
        t = wid + _NW * k

        @pl.when(t < _NB_FULL)
        def _():
            pltpu.sync_copy(table_t.at[:, pl.ds(t * 128, 128)], blk_v)

            def q_body(q, c):
                for half in range(2):
                    col = jnp.full((16,), 2 * q + half, jnp.int32)
                    for c0 in range(4):
                        rows = c0 * 16 + _iota16()
                        vals = plsc.load_gather(blk_v, [rows, col])
                        tr_v[q, pl.ds(half * 64 + c0 * 16, 16)] = vals
                return c

            lax.fori_loop(0, 64, q_body, 0)
            pltpu.sync_copy(tr_v, t2.at[pl.ds(t * 64, 64), :])

        return carry

    lax.fori_loop(0, (_NB_FULL + _NW - 1) // _NW, block_body, 0)

    @pl.when(wid == 0)
    def _():
        pltpu.sync_copy(edge, blk_v.at[pl.ds(0, 32), :])
        pltpu.sync_copy(blk_v.at[pl.ds(0, 32), :], t2.at[pl.ds(499968, 32), :])


_UNITS = 50 * 128
_PER_W = _UNITS // _NW


@functools.partial(
    pl.kernel,
    mesh=_mesh,
    out_type=jax.ShapeDtypeStruct((50, 8, 128, 8, 128), jnp.float32),
    compiler_params=pltpu.CompilerParams(
        use_tc_tiling_on_sc=False, needs_layout_passes=False),
    scratch_types=[
        pltpu.VMEM((128,), jnp.int32),
        pltpu.VMEM((128, 64), jnp.float32),
        pltpu.VMEM((64, 128), jnp.float32),
        pltpu.SemaphoreType.DMA,
    ],
)
def _gather(t_lin, idx, o5, iv, gv, tv, sem):
    wid = lax.axis_index("s") * 2 + lax.axis_index("c")

    def unit_body(k, carry):
        u = wid * _PER_W + k
        s = u // 128
        t = u % 128
        pltpu.sync_copy(idx.at[pl.ds(s * 16384 + t * 128, 128)], iv)
        pltpu.async_copy(t_lin.at[iv], gv, sem).wait()

        def d_body(d, c):
            col = jnp.full((16,), d, jnp.int32)
            for c0 in range(8):
                rows = c0 * 16 + _iota16()
                vals = plsc.load_gather(gv, [rows, col])
                tv[d, pl.ds(c0 * 16, 16)] = vals
            return c

        lax.fori_loop(0, 64, d_body, 0)
        for g in range(8):
            pltpu.sync_copy(tv.at[pl.ds(g * 8, 8), :], o5.at[s, g, t])
        return carry

    lax.fori_loop(0, _PER_W, unit_body, 0)


def kernel(batch_data, item_embeddings):
    table_t = item_embeddings.T
    edge = item_embeddings[_EDGE_ROWS:].reshape(32, 128)
    t2 = _transpose(table_t, edge)
    t_lin = t2.reshape(1000000, 64)
    idx_t = batch_data.T.reshape(-1).astype(jnp.int32)
    o5 = _gather(t_lin, idx_t)
    return o5.transpose(2, 4, 0, 1, 3).reshape(16384, 50, 64)


# speedup vs baseline: 1.2448x; 1.2448x over previous
import functools

import jax
import jax.numpy as jnp
from jax import lax
from jax.experimental import pallas as pl
from jax.experimental.pallas import tpu as pltpu
from jax.experimental.pallas import tpu_sc as plsc

_NW = 32
_NB_FULL = 7812
_EDGE_ROWS = _NB_FULL * 128

_mesh = plsc.VectorSubcoreMesh(core_axis_name="c", subcore_axis_name="s")


def _iota16():
    return lax.iota(jnp.int32, 16)


@functools.partial(
    pl.kernel,
    mesh=_mesh,
    out_type=jax.ShapeDtypeStruct((500000, 128), jnp.float32),
    compiler_params=pltpu.CompilerParams(needs_layout_passes=False),
    scratch_types=[
        pltpu.VMEM((64, 128), jnp.float32),
        pltpu.VMEM((64, 128), jnp.float32),
        pltpu.VMEM((64, 128), jnp.float32),
        pltpu.VMEM((64, 128), jnp.float32),
        pltpu.SemaphoreType.DMA,
        pltpu.SemaphoreType.DMA,
        pltpu.SemaphoreType.DMA,
        pltpu.SemaphoreType.DMA,
    ],
)
def _transpose(table_t, edge, t2, blk0, blk1, tr0, tr1,
               bs0, bs1, os0, os1):
    wid = lax.axis_index("s") * 2 + lax.axis_index("c")
    nblk = jnp.where(wid < _NB_FULL % _NW, _NB_FULL // _NW + 1, _NB_FULL // _NW)
    blks = (blk0, blk1)
    trs = (tr0, tr1)
    bsems = (bs0, bs1)
    osems = (os0, os1)
    rows4 = [c0 * 16 + _iota16() for c0 in range(4)]

    def issue_blk(t, p):
        pltpu.async_copy(table_t.at[:, pl.ds(t * 128, 128)], blks[p], bsems[p])

    def wait_blk(p):
        pltpu.make_async_copy(
            table_t.at[:, pl.ds(0, 128)], blks[p], bsems[p]).wait()

    def start_out(t, p):
        pltpu.async_copy(trs[p], t2.at[pl.ds(t * 64, 64), :], osems[p])

    def wait_out(p):
        pltpu.make_async_copy(
            trs[p], t2.at[pl.ds(0, 64), :], osems[p]).wait()

    def transpose_blk(blk, tr):
        def q_body(q, c):
            for half in range(2):
                col = jnp.full((16,), 2 * q + half, jnp.int32)
                for c0 in range(4):
                    vals = plsc.load_gather(blk, [rows4[c0], col])
                    tr[q, pl.ds(half * 64 + c0 * 16, 16)] = vals
            return c
        lax.fori_loop(0, 64, q_body, 0, unroll=4)

    issue_blk(wid, 0)

    def outer(m, carry):
        for p in range(2):
            k = 2 * m + p

            @pl.when(k < nblk)
            def _():
                @pl.when(k + 1 < nblk)
                def _():
                    issue_blk(wid + _NW * (k + 1), 1 - p)
                wait_blk(p)

                @pl.when(k >= 2)
                def _():
                    wait_out(p)
                transpose_blk(blks[p], trs[p])
                start_out(wid + _NW * k, p)
        return carry

    lax.fori_loop(0, (_NB_FULL // _NW + 2) // 2, outer, 0)
    wait_out(0)
    wait_out(1)

    @pl.when(wid == 0)
    def _():
        pltpu.sync_copy(edge, blk0.at[pl.ds(0, 32), :])
        pltpu.sync_copy(blk0.at[pl.ds(0, 32), :], t2.at[pl.ds(499968, 32), :])


_PER_W = 50 * 128 // _NW


@functools.partial(
    pl.kernel,
    mesh=_mesh,
    out_type=jax.ShapeDtypeStruct((50, 8, 128, 8, 128), jnp.float32),
    compiler_params=pltpu.CompilerParams(
        use_tc_tiling_on_sc=False, needs_layout_passes=False),
    scratch_types=[
        pltpu.VMEM((128,), jnp.int32),
        pltpu.VMEM((128,), jnp.int32),
        pltpu.VMEM((128, 64), jnp.float32),
        pltpu.VMEM((128, 64), jnp.float32),
        pltpu.VMEM((8, 8, 128), jnp.float32),
        pltpu.VMEM((8, 8, 128), jnp.float32),
        pltpu.SemaphoreType.DMA,
        pltpu.SemaphoreType.DMA,
        pltpu.SemaphoreType.DMA,
        pltpu.SemaphoreType.DMA,
    ],
)
def _gather(t_lin, idx, o5, iv0, iv1, gv0, gv1, tv0, tv1,
            gs0, gs1, os0, os1):
    wid = lax.axis_index("s") * 2 + lax.axis_index("c")
    ivs = (iv0, iv1)
    gvs = (gv0, gv1)
    tvs = (tv0, tv1)
    gsems = (gs0, gs1)
    osems = (os0, os1)
    rows8 = [c0 * 16 + _iota16() for c0 in range(8)]

    def issue_unit(u, p):
        s = u // 128
        t = u % 128
        pltpu.sync_copy(idx.at[pl.ds(s * 16384 + t * 128, 128)], ivs[p])
        pltpu.async_copy(t_lin.at[ivs[p]], gvs[p], gsems[p])

    def wait_g(p):
        pltpu.make_async_copy(t_lin.at[ivs[p]], gvs[p], gsems[p]).wait()

    def start_out(u, p):
        s = u // 128
        t = u % 128
        pltpu.async_copy(tvs[p], o5.at[s, :, t], osems[p])

    def wait_out(p):
        pltpu.make_async_copy(tvs[p], o5.at[0, :, 0], osems[p]).wait()

    def transpose_unit(gv, tv):
        def d_body(d, c):
            col = jnp.full((16,), d, jnp.int32)
            g = d // 8
            i = d % 8
            for c0 in range(8):
                vals = plsc.load_gather(gv, [rows8[c0], col])
                tv[g, i, pl.ds(c0 * 16, 16)] = vals
            return c
        lax.fori_loop(0, 64, d_body, 0, unroll=4)

    issue_unit(wid * _PER_W, 0)

    def outer(m, carry):
        for p in range(2):
            k = 2 * m + p
            u = wid * _PER_W + k

            @pl.when(k + 1 < _PER_W)
            def _():
                issue_unit(u + 1, 1 - p)
            wait_g(p)

            @pl.when(k >= 2)
            def _():
                wait_out(p)
            transpose_unit(gvs[p], tvs[p])
            start_out(u, p)
        return carry

    lax.fori_loop(0, _PER_W // 2, outer, 0)
    wait_out(0)
    wait_out(1)


def kernel(batch_data, item_embeddings):
    table_t = item_embeddings.T
    edge = item_embeddings[_EDGE_ROWS:].reshape(32, 128)
    t2 = _transpose(table_t, edge)
    t_lin = t2.reshape(1000000, 64)
    idx_t = batch_data.T.reshape(-1).astype(jnp.int32)
    o5 = _gather(t_lin, idx_t)
    return o5.transpose(2, 4, 0, 1, 3).reshape(16384, 50, 64)
